# Initial kernel scaffold; baseline (speedup 1.0000x reference)
import jax
import jax.numpy as jnp
from jax.experimental import pallas as pl


def kernel(x, edge_index, W1l, b1, W1r, gamma, beta, W2l, b2, W2r):
    raise NotImplementedError("write your pallas kernel here")


# speedup vs baseline: 4.0522x; 4.0522x over previous
import functools

import jax
import jax.numpy as jnp
from jax import lax
from jax.experimental import pallas as pl
from jax.experimental.pallas import tpu as pltpu
from jax.experimental.pallas import tpu_sc as plsc

N = 10000
E = 320000
D = 128
BN_EPS = 1e-5

NP = 10240
NSC = 2
NSUB = 16
NW = NSC * NSUB
CHUNK = 128
EPW = -(-E // (NW * CHUNK)) * CHUNK
EP = EPW * NW
RPS = NP // NSUB

BLK = 1024

_mesh = plsc.VectorSubcoreMesh(core_axis_name="c", subcore_axis_name="s")


@functools.partial(
    pl.kernel, mesh=_mesh,
    out_type=jax.ShapeDtypeStruct((NSC, NP, D), jnp.float32),
    scratch_types=[
        pltpu.VMEM((CHUNK,), jnp.int32),
        pltpu.VMEM((CHUNK,), jnp.int32),
        pltpu.VMEM((CHUNK, D), jnp.float32),
        pltpu.VMEM_SHARED((NP, D), jnp.float32),
        pltpu.SemaphoreType.DMA,
    ],
)
def _sc_segsum(y, srcp, dstp, z128, agg_out, src_v, dst_v, rows_v, agg_s, sem):
    cid = lax.axis_index("c")
    sid = lax.axis_index("s")
    sl = pl.ds(sid * RPS, RPS)
    pltpu.sync_copy(z128.at[sl], agg_s.at[sl])
    plsc.subcore_barrier()
    base = (cid * NSUB + sid) * EPW

    def loop(i, carry):
        off = pl.multiple_of(base + i * CHUNK, 8)
        pltpu.sync_copy(srcp.at[pl.ds(off, CHUNK)], src_v)
        pltpu.sync_copy(dstp.at[pl.ds(off, CHUNK)], dst_v)
        pltpu.async_copy(y.at[src_v], rows_v, sem).wait()
        pltpu.sync_copy(rows_v, agg_s.at[dst_v], add=True)
        return carry

    lax.fori_loop(0, EPW // CHUNK, loop, 0)
    plsc.subcore_barrier()
    pltpu.sync_copy(agg_s.at[sl], agg_out.at[cid, sl])


@functools.partial(
    pl.kernel, mesh=_mesh,
    compiler_params=pltpu.CompilerParams(needs_layout_passes=False),
    out_type=jax.ShapeDtypeStruct((NW, NP), jnp.float32),
    scratch_types=[
        pltpu.VMEM((CHUNK,), jnp.int32),
        pltpu.VMEM((NP,), jnp.float32),
    ],
)
def _sc_count(dstp, zrow, cnt_out, dst_v, cnt_loc):
    cid = lax.axis_index("c")
    sid = lax.axis_index("s")
    wid = cid * NSUB + sid
    pltpu.sync_copy(zrow, cnt_loc)
    base = wid * EPW
    ones = jnp.ones((16,), jnp.float32)

    def loop(i, carry):
        off = pl.multiple_of(base + i * CHUNK, 8)
        pltpu.sync_copy(dstp.at[pl.ds(off, CHUNK)], dst_v)
        for j in range(CHUNK // 16):
            idx = dst_v[pl.ds(j * 16, 16)]
            plsc.addupdate_scatter(cnt_loc, [idx], ones)
        return carry

    lax.fori_loop(0, EPW // CHUNK, loop, 0)
    pltpu.sync_copy(cnt_loc, cnt_out.at[wid])


def _dotT(a, w):
    return lax.dot_general(a, w, (((1,), (1,)), ((), ())),
                           preferred_element_type=jnp.float32)


def _tc_pre_body(x_ref, wl_ref, wr_ref, b_ref, y_ref, r_ref):
    xb = x_ref[...]
    y_ref[...] = _dotT(xb, wl_ref[...])
    r_ref[...] = _dotT(xb, wr_ref[...]) + b_ref[...]


def _tc_mid_body(a0_ref, a1_ref, cnt_ref, r1_ref, g_ref, be_ref,
                 wl_ref, wr_ref, b_ref, y_ref, r_ref):
    cnt = jnp.sum(cnt_ref[...], axis=0)[:, None]
    inv = 1.0 / jnp.maximum(cnt, 1.0)
    h = (a0_ref[...] + a1_ref[...]) * inv + r1_ref[...]
    h = h * g_ref[...] + be_ref[...]
    h = jnp.maximum(h, 0.0)
    y_ref[...] = _dotT(h, wl_ref[...])
    r_ref[...] = _dotT(h, wr_ref[...]) + b_ref[...]


def _tc_post_body(a0_ref, a1_ref, cnt_ref, r2_ref, o_ref):
    cnt = jnp.sum(cnt_ref[...], axis=0)[:, None]
    inv = 1.0 / jnp.maximum(cnt, 1.0)
    o_ref[...] = (a0_ref[...] + a1_ref[...]) * inv + r2_ref[...]


_row_spec = pl.BlockSpec((BLK, D), lambda i: (i, 0))
_cnt_spec = pl.BlockSpec((NW, BLK), lambda i: (0, i))
_full_spec = pl.BlockSpec((D, D), lambda i: (0, 0))
_vec_spec = pl.BlockSpec((1, D), lambda i: (0, 0))

_tc_pre = pl.pallas_call(
    _tc_pre_body,
    grid=(NP // BLK,),
    in_specs=[_row_spec, _full_spec, _full_spec, _vec_spec],
    out_specs=[_row_spec, _row_spec],
    out_shape=[jax.ShapeDtypeStruct((NP, D), jnp.float32)] * 2,
)

_tc_mid = pl.pallas_call(
    _tc_mid_body,
    grid=(NP // BLK,),
    in_specs=[_row_spec, _row_spec, _cnt_spec, _row_spec,
              _vec_spec, _vec_spec, _full_spec, _full_spec, _vec_spec],
    out_specs=[_row_spec, _row_spec],
    out_shape=[jax.ShapeDtypeStruct((NP, D), jnp.float32)] * 2,
)

_tc_post = pl.pallas_call(
    _tc_post_body,
    grid=(NP // BLK,),
    in_specs=[_row_spec, _row_spec, _cnt_spec, _row_spec],
    out_specs=_row_spec,
    out_shape=jax.ShapeDtypeStruct((NP, D), jnp.float32),
)


def kernel(x, edge_index, W1l, b1, W1r, gamma, beta, W2l, b2, W2r):
    src = edge_index[0]
    dst = edge_index[1]
    pad = EP - E
    srcp = jnp.concatenate([src, jnp.zeros((pad,), jnp.int32)])
    dstp = jnp.concatenate([dst, jnp.full((pad,), N, jnp.int32)])
    xp = jnp.pad(x, ((0, NP - N), (0, 0)))

    z128 = jnp.zeros((NP, D), jnp.float32)
    zrow = jnp.zeros((NP,), jnp.float32)

    g2 = gamma.reshape(1, D) / jnp.sqrt(1.0 + BN_EPS)
    be2 = beta.reshape(1, D)
    b1_2 = b1.reshape(1, D)
    b2_2 = b2.reshape(1, D)

    cnt32 = _sc_count(dstp, zrow)
    y1, r1 = _tc_pre(xp, W1l, W1r, b1_2)
    agg1 = _sc_segsum(y1, srcp, dstp, z128)
    y2, r2 = _tc_mid(agg1[0], agg1[1], cnt32, r1, g2, be2, W2l, W2r, b2_2)
    agg2 = _sc_segsum(y2, srcp, dstp, z128)
    out = _tc_post(agg2[0], agg2[1], cnt32, r2)
    return out[:N]
